# Initial kernel scaffold; baseline (speedup 1.0000x reference)
import jax
import jax.numpy as jnp
from jax.experimental import pallas as pl


def kernel(x, edge_index, W_self1, W_neigh1, b1, W_self2, W_neigh2, b2):
    raise NotImplementedError("write your pallas kernel here")


# speedup vs baseline: 4.5569x; 4.5569x over previous
import functools

import jax
import jax.numpy as jnp
from jax import lax
from jax.experimental import pallas as pl
from jax.experimental.pallas import tpu as pltpu
from jax.experimental.pallas import tpu_sc as plsc

N = 10000
E = 320000
D = 128
DP = 144
NPAD = 10240

NC = 2
NS = 16
NW = NC * NS
E_PER_W = E // NW
CHUNK = 80
NCHUNK = E_PER_W // CHUNK

ROWS_PER_SUB = NPAD // NS
DRAIN = 128
NDRAIN = ROWS_PER_SUB // DRAIN


def _sc_agg_body(x_hbm, src_hbm, dst_hbm, out_hbm,
                 idx_s, idx_d, rows, buf, acc, sem):
    c = lax.axis_index("c")
    s = lax.axis_index("s")
    wid = c * NS + s

    def _zero_row(r, _):
        for k in range(DP // 16):
            buf[r, pl.ds(k * 16, 16)] = jnp.zeros((16,), jnp.float32)
        return 0
    lax.fori_loop(0, DRAIN, _zero_row, 0)
    for j in range(NDRAIN):
        pltpu.sync_copy(buf, acc.at[pl.ds(s * ROWS_PER_SUB + j * DRAIN, DRAIN)])
    plsc.subcore_barrier()

    def _step(i, _):
        e0 = wid * E_PER_W + i * CHUNK
        pltpu.sync_copy(src_hbm.at[pl.ds(e0, CHUNK)], idx_s)
        pltpu.sync_copy(dst_hbm.at[pl.ds(e0, CHUNK)], idx_d)
        pltpu.async_copy(x_hbm.at[idx_s], rows, sem).wait()
        pltpu.sync_copy(rows, acc.at[idx_d], add=True)
        return 0
    lax.fori_loop(0, NCHUNK, _step, 0)
    plsc.subcore_barrier()

    for j in range(NDRAIN):
        r0 = s * ROWS_PER_SUB + j * DRAIN
        pltpu.sync_copy(acc.at[pl.ds(r0, DRAIN)], buf)
        pltpu.sync_copy(buf, out_hbm.at[pl.ds(c * NPAD + r0, DRAIN)])


_sc_agg = pl.kernel(
    _sc_agg_body,
    out_type=jax.ShapeDtypeStruct((2 * NPAD, DP), jnp.float32),
    mesh=plsc.VectorSubcoreMesh(core_axis_name="c", subcore_axis_name="s"),
    scratch_types=[
        pltpu.VMEM((CHUNK,), jnp.int32),
        pltpu.VMEM((CHUNK,), jnp.int32),
        pltpu.VMEM((CHUNK, DP), jnp.float32),
        pltpu.VMEM((DRAIN, DP), jnp.float32),
        pltpu.VMEM_SHARED((NPAD, DP), jnp.float32),
        pltpu.SemaphoreType.DMA,
    ],
    compiler_params=pltpu.CompilerParams(use_tc_tiling_on_sc=False),
)


def _tc_layer_body(x_ref, p_ref, ws_ref, wn_ref, b_ref, o_ref, *, relu, pad_out):
    p = p_ref[0] + p_ref[1]
    agg = p[:, :D]
    deg = p[:, D:D + 1]
    h_neigh = agg / jnp.maximum(deg, 1.0)
    y = (jnp.dot(x_ref[...][:, :D], ws_ref[...],
                 preferred_element_type=jnp.float32)
         + jnp.dot(h_neigh, wn_ref[...], preferred_element_type=jnp.float32)
         + b_ref[...])
    if relu:
        y = jnp.maximum(y, 0.0)
    if pad_out:
        o_ref[:, :D] = y
        pad = (lax.broadcasted_iota(jnp.int32, (y.shape[0], DP - D), 1) == 0)
        o_ref[:, D:] = pad.astype(jnp.float32)
    else:
        o_ref[...] = y


def _tc_layer(x_pad, partials, W_self, W_neigh, b, *, relu, pad_out):
    BN = 1000
    grid = (N // BN,)
    out_w = DP if pad_out else D
    return pl.pallas_call(
        functools.partial(_tc_layer_body, relu=relu, pad_out=pad_out),
        grid=grid,
        in_specs=[
            pl.BlockSpec((BN, DP), lambda i: (i, 0)),
            pl.BlockSpec((2, BN, DP), lambda i: (0, i, 0)),
            pl.BlockSpec((D, D), lambda i: (0, 0)),
            pl.BlockSpec((D, D), lambda i: (0, 0)),
            pl.BlockSpec((1, D), lambda i: (0, 0)),
        ],
        out_specs=pl.BlockSpec((BN, out_w), lambda i: (i, 0)),
        out_shape=jax.ShapeDtypeStruct((N, out_w), jnp.float32),
        compiler_params=pltpu.CompilerParams(
            dimension_semantics=("parallel",)),
    )(x_pad, partials, W_self, W_neigh, b)


def kernel(x, edge_index, W_self1, W_neigh1, b1, W_self2, W_neigh2, b2):
    ei = edge_index.astype(jnp.int32)
    src, dst = ei[0], ei[1]
    x_pad = jnp.concatenate(
        [x, jnp.ones((N, 1), jnp.float32), jnp.zeros((N, DP - D - 1), jnp.float32)],
        axis=1)
    p1 = _sc_agg(x_pad, src, dst).reshape(2, NPAD, DP)
    h1 = _tc_layer(x_pad, p1, W_self1, W_neigh1, b1.reshape(1, D),
                   relu=True, pad_out=True)
    p2 = _sc_agg(h1, src, dst).reshape(2, NPAD, DP)
    return _tc_layer(h1, p2, W_self2, W_neigh2, b2.reshape(1, D),
                     relu=False, pad_out=False)
